# Initial kernel scaffold; baseline (speedup 1.0000x reference)
import jax
import jax.numpy as jnp
from jax.experimental import pallas as pl


def kernel(user_table, item_table, u_w0, i_w0, u_w1, i_w1, u_cat_w, i_cat_w, edge_src_b0, edge_dst_b0, edge_val_b0, edge_src_b1, edge_dst_b1, edge_val_b1):
    raise NotImplementedError("write your pallas kernel here")


# speedup vs baseline: 1.9407x; 1.9407x over previous
import functools

import jax
import jax.numpy as jnp
from jax import lax
from jax.experimental import pallas as pl
from jax.experimental.pallas import tpu as pltpu
from jax.experimental.pallas import tpu_sc as plsc

U_NUM = 50000
I_NUM = 10000
DIM = 128
E_NUM = 500000

NC = 2
NS = 16
NW = NC * NS
EP = 512000
EW = EP // NW
OP = 128
NOPS = EW // OP
CG = 4
CW = DIM // CG


def _make_spmm(n_tbl_rows, n_out):
  ZR = 200
  nch = n_out // ZR
  kmax = (nch + NS - 1) // NS

  def body(tbl4, gidx4, sidx, val, out, acc, zbuf, rows, gib, sib, vbuf,
           gsem0, gsem1):
    cid = lax.axis_index("c")
    sid = lax.axis_index("s")
    wid = sid * NC + cid
    ebase = wid * EW

    @pl.loop(0, ZR)
    def _fill(e):
      zbuf[e, pl.ds(0, 16)] = jnp.zeros((16,), jnp.float32)
      zbuf[e, pl.ds(16, 16)] = jnp.zeros((16,), jnp.float32)

    def load_fire(op_idx, slot, cg, gsem):
      base = ebase + op_idx * OP
      pltpu.sync_copy(gidx4.at[cg, pl.ds(base, OP)], gib.at[slot])
      pltpu.sync_copy(sidx.at[pl.ds(base, OP)], sib.at[slot])
      pltpu.sync_copy(val.at[pl.ds(base, OP)], vbuf.at[pl.ds(slot * OP, OP)])
      pltpu.async_copy(tbl4.at[gib.at[slot]], rows.at[slot], gsem)

    def wait_g(slot, gsem):
      pltpu.make_async_copy(tbl4.at[gib.at[slot]], rows.at[slot], gsem).wait()

    def scale(slot):
      @plsc.parallel_loop(0, OP // 16, unroll=2)
      def _s(g):
        vvec = vbuf[pl.ds(slot * OP + g * 16, 16)]
        for l in range(16):
          vv = lax.broadcast(vvec[l], (16,))
          e = g * 16 + l
          rows[slot, e, pl.ds(0, 16)] = rows[slot, e, pl.ds(0, 16)] * vv
          rows[slot, e, pl.ds(16, 16)] = rows[slot, e, pl.ds(16, 16)] * vv

    def scatter(slot):
      pltpu.sync_copy(rows.at[slot], acc.at[sib.at[slot]], add=True)

    @pl.loop(0, CG)
    def _pass(cg):
      for k in range(kmax):
        ch = sid + NS * k

        @pl.when(ch < nch)
        def _z():
          pltpu.sync_copy(zbuf, acc.at[pl.ds(ch * ZR, ZR)])

      plsc.subcore_barrier()

      load_fire(0, 0, cg, gsem0)

      @pl.loop(0, (NOPS - 1) // 2)
      def _t(t):
        load_fire(2 * t + 1, 1, cg, gsem1)
        wait_g(0, gsem0)
        scale(0)
        scatter(0)
        load_fire(2 * t + 2, 0, cg, gsem0)
        wait_g(1, gsem1)
        scale(1)
        scatter(1)

      wait_g(0, gsem0)
      scale(0)
      scatter(0)
      plsc.subcore_barrier()

      for k in range(kmax):
        ch = sid + NS * k

        @pl.when(ch < nch)
        def _c():
          pltpu.sync_copy(acc.at[pl.ds(ch * ZR, ZR)],
                          out.at[cid, cg, pl.ds(ch * ZR, ZR)])

      plsc.subcore_barrier()

  return pl.kernel(
      body,
      out_type=jax.ShapeDtypeStruct((NC, CG, n_out, CW), jnp.float32),
      mesh=plsc.VectorSubcoreMesh(core_axis_name="c", subcore_axis_name="s"),
      compiler_params=pltpu.CompilerParams(use_tc_tiling_on_sc=False),
      scratch_types=[
          pltpu.VMEM_SHARED((n_out, CW), jnp.float32),
          pltpu.VMEM((ZR, CW), jnp.float32),
          pltpu.VMEM((2, OP, CW), jnp.float32),
          pltpu.VMEM((2, OP), jnp.int32),
          pltpu.VMEM((2, OP), jnp.int32),
          pltpu.VMEM((2 * OP,), jnp.float32),
          pltpu.SemaphoreType.DMA,
          pltpu.SemaphoreType.DMA,
      ],
  )


_spmm_u = _make_spmm(CG * I_NUM, U_NUM)
_spmm_i = _make_spmm(CG * U_NUM, I_NUM)


def _dense_body(tbl_ref, p4_ref, w0_ref, w1_ref, cw0_ref, cw1_ref, uep_ref,
                ue_ref, nxt_ref, nxt4_ref, mean_ref, *, first):
  mparts = [p4_ref[0, c] + p4_ref[1, c] for c in range(CG)]
  msg = jnp.concatenate(mparts, axis=-1)
  tbl = tbl_ref[...]
  x = tbl + msg
  e0 = jax.nn.sigmoid(jnp.dot(x, w0_ref[...], preferred_element_type=jnp.float32))
  e1 = jax.nn.sigmoid(jnp.dot(x, w1_ref[...], preferred_element_type=jnp.float32))
  ue = (jnp.dot(e0, cw0_ref[...], preferred_element_type=jnp.float32)
        + jnp.dot(e1, cw1_ref[...], preferred_element_type=jnp.float32))
  ue_ref[...] = ue
  if first:
    nxt = tbl + ue
    nxt_ref[...] = nxt
    for c in range(CG):
      nxt4_ref[c] = nxt[:, c * CW:(c + 1) * CW]
  else:
    mean_ref[...] = (ue + uep_ref[...]) * 0.5


def _dense(tbl, p4, w0, w1, cat_w, ue_prev, first):
  n = tbl.shape[0]
  bs = 1000
  grid = (n // bs,)
  cw0 = cat_w[:DIM]
  cw1 = cat_w[DIM:]
  row_spec = pl.BlockSpec((bs, DIM), lambda i: (i, 0))
  w_spec = pl.BlockSpec((DIM, DIM), lambda i: (0, 0))
  out_shapes = [jax.ShapeDtypeStruct((n, DIM), jnp.float32)]
  out_specs = [row_spec]
  if first:
    out_shapes += [jax.ShapeDtypeStruct((n, DIM), jnp.float32),
                   jax.ShapeDtypeStruct((CG, n, CW), jnp.float32)]
    out_specs += [row_spec, pl.BlockSpec((CG, bs, CW), lambda i: (0, i, 0))]
  else:
    out_shapes += [jax.ShapeDtypeStruct((n, DIM), jnp.float32)]
    out_specs += [row_spec]

  def kbody(tbl_ref, p4_ref, w0_ref, w1_ref, cw0_ref, cw1_ref, uep_ref, *outs):
    if first:
      ue_ref, nxt_ref, nxt4_ref = outs
      mean_ref = None
    else:
      ue_ref, mean_ref = outs
      nxt_ref = nxt4_ref = None
    _dense_body(tbl_ref, p4_ref, w0_ref, w1_ref, cw0_ref, cw1_ref, uep_ref,
                ue_ref, nxt_ref, nxt4_ref, mean_ref, first=first)

  return pl.pallas_call(
      kbody,
      grid=grid,
      in_specs=[
          row_spec,
          pl.BlockSpec((NC, CG, bs, CW), lambda i: (0, 0, i, 0)),
          w_spec, w_spec, w_spec, w_spec,
          row_spec,
      ],
      out_specs=out_specs,
      out_shape=out_shapes,
  )(tbl, p4, w0, w1, cw0, cw1, ue_prev)


def _repack(tbl):
  n = tbl.shape[0]
  return tbl.reshape(n, CG, CW).transpose(1, 0, 2).reshape(CG * n, CW)


def _prep_edges(src, dst, val):
  pad = EP - E_NUM
  src = jnp.concatenate([src.astype(jnp.int32), jnp.zeros((pad,), jnp.int32)])
  dst = jnp.concatenate([dst.astype(jnp.int32), jnp.zeros((pad,), jnp.int32)])
  val = jnp.concatenate([val, jnp.zeros((pad,), jnp.float32)])
  src4 = src[None, :] + (jnp.arange(CG, dtype=jnp.int32) * U_NUM)[:, None]
  dst4 = dst[None, :] + (jnp.arange(CG, dtype=jnp.int32) * I_NUM)[:, None]
  return src, dst, val, src4, dst4


def kernel(user_table, item_table, u_w0, i_w0, u_w1, i_w1, u_cat_w, i_cat_w,
           edge_src_b0, edge_dst_b0, edge_val_b0,
           edge_src_b1, edge_dst_b1, edge_val_b1):
  src0, dst0, val0, src4_0, dst4_0 = _prep_edges(edge_src_b0, edge_dst_b0,
                                                 edge_val_b0)
  src1, dst1, val1, src4_1, dst4_1 = _prep_edges(edge_src_b1, edge_dst_b1,
                                                 edge_val_b1)

  u4 = _repack(user_table)
  i4 = _repack(item_table)

  up0 = _spmm_u(i4, dst4_0, src0, val0)
  ip0 = _spmm_i(u4, src4_0, dst0, val0)

  dummy_u = user_table
  dummy_i = item_table
  ue0, nu, nu4 = _dense(user_table, up0, u_w0, u_w1, u_cat_w, dummy_u, True)
  ie0, ni, ni4 = _dense(item_table, ip0, i_w0, i_w1, i_cat_w, dummy_i, True)

  up1 = _spmm_u(ni4.reshape(CG * I_NUM, CW), dst4_1, src1, val1)
  ip1 = _spmm_i(nu4.reshape(CG * U_NUM, CW), src4_1, dst1, val1)

  ue1, u_mean = _dense(nu, up1, u_w0, u_w1, u_cat_w, ue0, False)
  ie1, i_mean = _dense(ni, ip1, i_w0, i_w1, i_cat_w, ie0, False)

  return (u_mean, i_mean,
          jnp.stack([ue0, ue1], axis=0),
          jnp.stack([ie0, ie1], axis=0))
